# baseline (speedup 1.0000x reference)
import jax
import jax.numpy as jnp
from jax import lax
from jax.experimental import pallas as pl
from jax.experimental.pallas import tpu as pltpu
from jax.experimental.pallas import tpu_sc as plsc

N_USERS = 1000000
N_MOVIES = 100000
N_FACTORS = 128
BATCH = 16384

NC = 2
NS = 16
L = 16
NW = NC * NS
PER_W = BATCH // NW
CHUNK = 128
NCHUNK = PER_W // CHUNK


def _sc_body(users_hbm, movies_hbm, ut_hbm, mt_hbm, ubt_hbm, mbt_hbm,
             gb_hbm, out_hbm,
             uidx, midx, uq, mq, urows, mrows, ubrows, mbrows, gbv, out_v,
             sem0, sem1, sem2, sem3):
    wid = lax.axis_index("s") * NC + lax.axis_index("c")
    base = wid * PER_W

    pltpu.sync_copy(gb_hbm, gbv.at[pl.ds(0, 1)])
    gb_vec = jnp.broadcast_to(gbv[:][0], (L,))

    zeros16 = jnp.zeros((L,), jnp.int32)
    iota16 = lax.iota(jnp.int32, L)

    for c in range(NCHUNK):
        cbase = base + c * CHUNK
        pltpu.sync_copy(users_hbm.at[pl.ds(cbase, CHUNK)], uidx)
        pltpu.sync_copy(movies_hbm.at[pl.ds(cbase, CHUNK)], midx)
        for g in range(CHUNK // L):
            sl = pl.ds(g * L, L)
            uq[sl] = lax.shift_right_logical(uidx[sl], 4)
            mq[sl] = lax.shift_right_logical(midx[sl], 4)
        cp0 = pltpu.async_copy(ut_hbm.at[uidx], urows, sem0)
        cp1 = pltpu.async_copy(mt_hbm.at[midx], mrows, sem1)
        cp2 = pltpu.async_copy(ubt_hbm.at[uq], ubrows, sem2)
        cp3 = pltpu.async_copy(mbt_hbm.at[mq], mbrows, sem3)
        cp0.wait()
        cp1.wait()
        cp2.wait()
        cp3.wait()

        for g in range(CHUNK // L):
            items = iota16 + (g * L)

            def fbody(f, acc):
                fv = zeros16 + f
                u = plsc.load_gather(urows, [items, fv])
                m = plsc.load_gather(mrows, [items, fv])
                return acc + u * m

            acc = lax.fori_loop(0, N_FACTORS, fbody,
                                jnp.zeros((L,), jnp.float32))
            sl = pl.ds(g * L, L)
            ucol = jnp.bitwise_and(uidx[sl], 15)
            mcol = jnp.bitwise_and(midx[sl], 15)
            ubias = plsc.load_gather(ubrows, [items, ucol])
            mbias = plsc.load_gather(mbrows, [items, mcol])
            res = acc + ubias + mbias + gb_vec
            out_v[pl.ds(c * CHUNK + g * L, L)] = res

    pltpu.sync_copy(out_v, out_hbm.at[pl.ds(base, PER_W)])


@jax.jit
def kernel(users, movies, user_table, movie_table, user_bias_table,
           movie_bias_table, global_bias):
    users = users.astype(jnp.int32)
    movies = movies.astype(jnp.int32)
    ubt16 = user_bias_table.reshape(N_USERS // L, L)
    mbt16 = movie_bias_table.reshape(N_MOVIES // L, L)

    mesh = plsc.VectorSubcoreMesh(core_axis_name="c", subcore_axis_name="s")
    run = pl.kernel(
        _sc_body,
        out_type=jax.ShapeDtypeStruct((BATCH,), jnp.float32),
        mesh=mesh,
        compiler_params=pltpu.CompilerParams(
            needs_layout_passes=False, use_tc_tiling_on_sc=False),
        scratch_types=[
            pltpu.VMEM((CHUNK,), jnp.int32),
            pltpu.VMEM((CHUNK,), jnp.int32),
            pltpu.VMEM((CHUNK,), jnp.int32),
            pltpu.VMEM((CHUNK,), jnp.int32),
            pltpu.VMEM((CHUNK, N_FACTORS), jnp.float32),
            pltpu.VMEM((CHUNK, N_FACTORS), jnp.float32),
            pltpu.VMEM((CHUNK, L), jnp.float32),
            pltpu.VMEM((CHUNK, L), jnp.float32),
            pltpu.VMEM((L,), jnp.float32),
            pltpu.VMEM((PER_W,), jnp.float32),
            pltpu.SemaphoreType.DMA,
            pltpu.SemaphoreType.DMA,
            pltpu.SemaphoreType.DMA,
            pltpu.SemaphoreType.DMA,
        ],
    )
    return run(users, movies, user_table, movie_table, ubt16,
               mbt16, global_bias)
